# baseline (speedup 1.0000x reference)
import functools

import jax
import jax.numpy as jnp
from jax import lax
from jax.experimental import pallas as pl
from jax.experimental.pallas import tpu as pltpu
from jax.experimental.pallas import tpu_sc as plsc

_K = 1024
_DIM = 50257
_Q = 32
_D_BLK = 2048
_L = 16
_NCHUNK = _K // _L


def _kl_body(q_ref, at_ref, out_ref, ent_acc, cross_acc):
    j = pl.program_id(0)
    nd = pl.num_programs(0)

    @pl.when(j == 0)
    def _init():
        ent_acc[...] = jnp.zeros_like(ent_acc)
        cross_acc[...] = jnp.zeros_like(cross_acc)

    at = at_ref[...]
    q = q_ref[...]
    rem = _DIM - j * _D_BLK
    rowm = jax.lax.broadcasted_iota(jnp.int32, (_D_BLK, 1), 0) < rem
    colm = jax.lax.broadcasted_iota(jnp.int32, (1, _D_BLK), 1) < rem
    a_m = jnp.where(rowm, at, 1.0)
    lq = jnp.where(colm, jnp.log(q), 0.0)
    al = a_m * jnp.log(a_m)
    ones = jnp.ones((1, _D_BLK), jnp.float32)
    ent_acc[...] += jax.lax.dot_general(
        ones, al, (((1,), (0,)), ((), ())),
        preferred_element_type=jnp.float32)
    cross_acc[...] += jax.lax.dot_general(
        lq, a_m, (((1,), (0,)), ((), ())),
        preferred_element_type=jnp.float32)

    @pl.when(j == nd - 1)
    def _finish():
        out_ref[...] = ent_acc[...] / _DIM - cross_acc[...] / _DIM


def _kl_matrix(query, at):
    nd = (_DIM + _D_BLK - 1) // _D_BLK
    return pl.pallas_call(
        _kl_body,
        grid=(nd,),
        in_specs=[
            pl.BlockSpec((_Q, _D_BLK), lambda j: (0, j)),
            pl.BlockSpec((_D_BLK, _K), lambda j: (j, 0)),
        ],
        out_specs=pl.BlockSpec((_Q, _K), lambda j: (0, 0)),
        out_shape=jax.ShapeDtypeStruct((_Q, _K), jnp.float32),
        scratch_shapes=[
            pltpu.VMEM((1, _K), jnp.float32),
            pltpu.VMEM((_Q, _K), jnp.float32),
        ],
        compiler_params=pltpu.CompilerParams(
            dimension_semantics=("arbitrary",)),
    )(query, at)


_DNUMS = lax.GatherDimensionNumbers(
    offset_dims=(), collapsed_slice_dims=(0,), start_index_map=(0,))


def _shuffle(v, perm):
    return lax.gather(v, perm[:, None], _DNUMS, (1,),
                      mode=lax.GatherScatterMode.PROMISE_IN_BOUNDS)


def _bfly(v, op, iota):
    for shift in (8, 4, 2, 1):
        v = op(v, _shuffle(v, jnp.bitwise_xor(iota, shift)))
    return v


def _sc_select_body(kl_hbm, lab_hbm, out_hbm, klrow, labv, res16):
    wid = lax.axis_index("s") * 2 + lax.axis_index("c")
    pltpu.sync_copy(kl_hbm.at[wid], klrow)
    pltpu.sync_copy(lab_hbm, labv)

    iota = lax.iota(jnp.int32, _L)

    def _min_step(c, m):
        return jnp.minimum(m, klrow[pl.ds(c * _L, _L)])

    m16 = lax.fori_loop(0, _NCHUNK, _min_step,
                        jnp.full((_L,), jnp.inf, jnp.float32))
    minv = _bfly(m16, jnp.minimum, iota)

    def _idx_step(c, best):
        v = klrow[pl.ds(c * _L, _L)]
        cand = jnp.where(v == minv, iota + c * _L, _K)
        return jnp.minimum(best, cand)

    idx16 = lax.fori_loop(0, _NCHUNK, _idx_step,
                          jnp.full((_L,), _K, jnp.int32))
    idxv = _bfly(idx16, jnp.minimum, iota)

    def _lab_step(c, acc):
        lc = labv[pl.ds(c * _L, _L)]
        return jnp.maximum(acc, jnp.where(iota + c * _L == idxv, lc, 0))

    lab16 = lax.fori_loop(0, _NCHUNK, _lab_step,
                          jnp.zeros((_L,), jnp.int32))
    res16[...] = _bfly(lab16, jnp.maximum, iota)
    pltpu.sync_copy(res16, out_hbm.at[wid])


def _sc_select(kl, queue_label):
    mesh = plsc.VectorSubcoreMesh(core_axis_name="c", subcore_axis_name="s")
    k = functools.partial(
        pl.kernel,
        mesh=mesh,
        out_type=jax.ShapeDtypeStruct((_Q, _L), jnp.int32),
        scratch_types=[
            pltpu.VMEM((_K,), jnp.float32),
            pltpu.VMEM((_K,), jnp.int32),
            pltpu.VMEM((_L,), jnp.int32),
        ],
    )(_sc_select_body)
    return k(kl, queue_label)


@jax.jit
def kernel(query, queue_anchor, queue_label):
    at = queue_anchor.T
    kl = _kl_matrix(query, at)
    return _sc_select(kl, queue_label)[:, 0]


# speedup vs baseline: 1.0249x; 1.0249x over previous
import functools

import jax
import jax.numpy as jnp
from jax import lax
from jax.experimental import pallas as pl
from jax.experimental.pallas import tpu as pltpu
from jax.experimental.pallas import tpu_sc as plsc

_K = 1024
_DIM = 50257
_Q = 32
_D_BLK = 3072
_L = 16
_NCHUNK = _K // _L


def _kl_body(q_ref, at_ref, out_ref, ent_acc, cross_acc):
    j = pl.program_id(0)
    nd = pl.num_programs(0)

    @pl.when(j == 0)
    def _init():
        ent_acc[...] = jnp.zeros_like(ent_acc)
        cross_acc[...] = jnp.zeros_like(cross_acc)

    at = at_ref[...]
    q = q_ref[...]
    rem = _DIM - j * _D_BLK
    rowm = jax.lax.broadcasted_iota(jnp.int32, (_D_BLK, 1), 0) < rem
    colm = jax.lax.broadcasted_iota(jnp.int32, (1, _D_BLK), 1) < rem
    a_m = jnp.where(rowm, at, 1.0)
    lq = jnp.where(colm, jnp.log(q), 0.0)
    al = a_m * jnp.log(a_m)
    ones = jnp.ones((1, _D_BLK), jnp.float32)
    ent_acc[...] += jax.lax.dot_general(
        ones, al, (((1,), (0,)), ((), ())),
        preferred_element_type=jnp.float32)
    cross_acc[...] += jax.lax.dot_general(
        lq, a_m, (((1,), (0,)), ((), ())),
        preferred_element_type=jnp.float32)

    @pl.when(j == nd - 1)
    def _finish():
        out_ref[...] = ent_acc[...] / _DIM - cross_acc[...] / _DIM


def _kl_matrix(query, at):
    nd = (_DIM + _D_BLK - 1) // _D_BLK
    return pl.pallas_call(
        _kl_body,
        grid=(nd,),
        in_specs=[
            pl.BlockSpec((_Q, _D_BLK), lambda j: (0, j)),
            pl.BlockSpec((_D_BLK, _K), lambda j: (j, 0)),
        ],
        out_specs=pl.BlockSpec((_Q, _K), lambda j: (0, 0)),
        out_shape=jax.ShapeDtypeStruct((_Q, _K), jnp.float32),
        scratch_shapes=[
            pltpu.VMEM((1, _K), jnp.float32),
            pltpu.VMEM((_Q, _K), jnp.float32),
        ],
        compiler_params=pltpu.CompilerParams(
            dimension_semantics=("arbitrary",)),
    )(query, at)


_DNUMS = lax.GatherDimensionNumbers(
    offset_dims=(), collapsed_slice_dims=(0,), start_index_map=(0,))


def _shuffle(v, perm):
    return lax.gather(v, perm[:, None], _DNUMS, (1,),
                      mode=lax.GatherScatterMode.PROMISE_IN_BOUNDS)


def _bfly(v, op, iota):
    for shift in (8, 4, 2, 1):
        v = op(v, _shuffle(v, jnp.bitwise_xor(iota, shift)))
    return v


def _sc_select_body(kl_hbm, lab_hbm, out_hbm, klrow, labv, res16):
    wid = lax.axis_index("s") * 2 + lax.axis_index("c")
    pltpu.sync_copy(kl_hbm.at[wid], klrow)
    pltpu.sync_copy(lab_hbm, labv)

    iota = lax.iota(jnp.int32, _L)

    def _min_step(c, m):
        return jnp.minimum(m, klrow[pl.ds(c * _L, _L)])

    m16 = lax.fori_loop(0, _NCHUNK, _min_step,
                        jnp.full((_L,), jnp.inf, jnp.float32))
    minv = _bfly(m16, jnp.minimum, iota)

    def _idx_step(c, best):
        v = klrow[pl.ds(c * _L, _L)]
        cand = jnp.where(v == minv, iota + c * _L, _K)
        return jnp.minimum(best, cand)

    idx16 = lax.fori_loop(0, _NCHUNK, _idx_step,
                          jnp.full((_L,), _K, jnp.int32))
    idxv = _bfly(idx16, jnp.minimum, iota)

    def _lab_step(c, acc):
        lc = labv[pl.ds(c * _L, _L)]
        return jnp.maximum(acc, jnp.where(iota + c * _L == idxv, lc, 0))

    lab16 = lax.fori_loop(0, _NCHUNK, _lab_step,
                          jnp.zeros((_L,), jnp.int32))
    res16[...] = _bfly(lab16, jnp.maximum, iota)
    pltpu.sync_copy(res16, out_hbm.at[wid])


def _sc_select(kl, queue_label):
    mesh = plsc.VectorSubcoreMesh(core_axis_name="c", subcore_axis_name="s")
    k = functools.partial(
        pl.kernel,
        mesh=mesh,
        out_type=jax.ShapeDtypeStruct((_Q, _L), jnp.int32),
        scratch_types=[
            pltpu.VMEM((_K,), jnp.float32),
            pltpu.VMEM((_K,), jnp.int32),
            pltpu.VMEM((_L,), jnp.int32),
        ],
    )(_sc_select_body)
    return k(kl, queue_label)


@jax.jit
def kernel(query, queue_anchor, queue_label):
    at = queue_anchor.T
    kl = _kl_matrix(query, at)
    return _sc_select(kl, queue_label)[:, 0]


# speedup vs baseline: 1.0314x; 1.0064x over previous
import functools

import jax
import jax.numpy as jnp
from jax import lax
from jax.experimental import pallas as pl
from jax.experimental.pallas import tpu as pltpu
from jax.experimental.pallas import tpu_sc as plsc

_K = 1024
_DIM = 50257
_Q = 32
_D_BLK = 3072
_L = 16
_NCHUNK = _K // _L


def _kl_body(q_ref, at_ref, out_ref, ent_acc, cross_acc):
    j = pl.program_id(0)
    nd = pl.num_programs(0)

    @pl.when(j == 0)
    def _init():
        ent_acc[...] = jnp.zeros_like(ent_acc)
        cross_acc[...] = jnp.zeros_like(cross_acc)

    at = at_ref[...]
    q = q_ref[...]
    rem = _DIM - j * _D_BLK
    rowm = jax.lax.broadcasted_iota(jnp.int32, (_D_BLK, 1), 0) < rem
    colm = jax.lax.broadcasted_iota(jnp.int32, (1, _D_BLK), 1) < rem
    a_m = jnp.where(rowm, at, 1.0)
    lq = jnp.where(colm, jnp.log(q), 0.0)
    al = a_m * jnp.log(a_m)
    ones = jnp.ones((1, _D_BLK), jnp.float32)
    ent_acc[...] += jax.lax.dot_general(
        ones, al, (((1,), (0,)), ((), ())),
        preferred_element_type=jnp.float32)
    cross_acc[...] += jax.lax.dot_general(
        lq, a_m, (((1,), (0,)), ((), ())),
        preferred_element_type=jnp.float32)

    @pl.when(j == nd - 1)
    def _finish():
        out_ref[...] = ent_acc[...] / _DIM - cross_acc[...] / _DIM


def _kl_matrix(query, at):
    nd = (_DIM + _D_BLK - 1) // _D_BLK
    return pl.pallas_call(
        _kl_body,
        grid=(nd,),
        in_specs=[
            pl.BlockSpec((_Q, _D_BLK), lambda j: (0, j)),
            pl.BlockSpec((_D_BLK, _K), lambda j: (j, 0)),
        ],
        out_specs=pl.BlockSpec((_Q, _K), lambda j: (0, 0)),
        out_shape=jax.ShapeDtypeStruct((_Q, _K), jnp.float32),
        scratch_shapes=[
            pltpu.VMEM((1, _K), jnp.float32),
            pltpu.VMEM((_Q, _K), jnp.float32),
        ],
        compiler_params=pltpu.CompilerParams(
            dimension_semantics=("arbitrary",)),
    )(query, at)


_DNUMS = lax.GatherDimensionNumbers(
    offset_dims=(), collapsed_slice_dims=(0,), start_index_map=(0,))


def _shuffle(v, perm):
    return lax.gather(v, perm[:, None], _DNUMS, (1,),
                      mode=lax.GatherScatterMode.PROMISE_IN_BOUNDS)


def _bfly(v, op, iota):
    for shift in (8, 4, 2, 1):
        v = op(v, _shuffle(v, jnp.bitwise_xor(iota, shift)))
    return v


def _sc_select_body(kl_hbm, lab_hbm, out_hbm, klrow, labv, res16):
    wid = lax.axis_index("s") * 2 + lax.axis_index("c")
    pltpu.sync_copy(kl_hbm.at[wid], klrow)
    pltpu.sync_copy(lab_hbm, labv)

    iota = lax.iota(jnp.int32, _L)

    def _pair_step(c, carry):
        m, mi = carry
        v = klrow[pl.ds(c * _L, _L)]
        better = v < m
        return (jnp.where(better, v, m),
                jnp.where(better, iota + c * _L, mi))

    m16, mi16 = lax.fori_loop(
        0, _NCHUNK, _pair_step,
        (jnp.full((_L,), jnp.inf, jnp.float32),
         jnp.full((_L,), _K, jnp.int32)),
        unroll=8)

    for shift in (8, 4, 2, 1):
        perm = jnp.bitwise_xor(iota, shift)
        pm, pmi = _shuffle(m16, perm), _shuffle(mi16, perm)
        better = (pm < m16) | ((pm == m16) & (pmi < mi16))
        m16 = jnp.where(better, pm, m16)
        mi16 = jnp.where(better, pmi, mi16)
    idxv = mi16

    def _lab_step(c, acc):
        lc = labv[pl.ds(c * _L, _L)]
        return jnp.maximum(acc, jnp.where(iota + c * _L == idxv, lc, 0))

    lab16 = lax.fori_loop(0, _NCHUNK, _lab_step,
                          jnp.zeros((_L,), jnp.int32), unroll=8)
    res16[...] = _bfly(lab16, jnp.maximum, iota)
    pltpu.sync_copy(res16, out_hbm.at[wid])


def _sc_select(kl, queue_label):
    mesh = plsc.VectorSubcoreMesh(core_axis_name="c", subcore_axis_name="s")
    k = functools.partial(
        pl.kernel,
        mesh=mesh,
        out_type=jax.ShapeDtypeStruct((_Q, _L), jnp.int32),
        scratch_types=[
            pltpu.VMEM((_K,), jnp.float32),
            pltpu.VMEM((_K,), jnp.int32),
            pltpu.VMEM((_L,), jnp.int32),
        ],
    )(_sc_select_body)
    return k(kl, queue_label)


@jax.jit
def kernel(query, queue_anchor, queue_label):
    at = queue_anchor.T
    kl = _kl_matrix(query, at)
    return _sc_select(kl, queue_label)[:, 0]
